# baseline (speedup 1.0000x reference)
import functools

import jax
import jax.numpy as jnp
from jax import lax
from jax.experimental import pallas as pl
from jax.experimental.pallas import tpu as pltpu
from jax.experimental.pallas import tpu_sc as plsc

N = 10000
F = 128
E = 320000
NCLASS = 40

NC = 2
NS = 16
NW = NC * NS

CHUNK = 64
CPW = 160
NHALF = 4
HALF = CPW // NHALF
NBUF = 4
EPW = CPW * CHUNK
EPAD = NW * EPW
ZROWS = 10112
RPS = ZROWS // NS

BLK = 1000

_P = lax.Precision.HIGHEST


def _sc_agg(h, src_w, dst_w, zeros_rows):
    mesh = plsc.VectorSubcoreMesh(core_axis_name="c", subcore_axis_name="s")

    @functools.partial(
        pl.kernel,
        out_type=jax.ShapeDtypeStruct((NC, ZROWS, F), jnp.float32),
        mesh=mesh,
        scratch_types=[
            pltpu.VMEM((HALF, CHUNK), jnp.int32),
            pltpu.VMEM((HALF, CHUNK), jnp.int32),
            pltpu.VMEM((NBUF, CHUNK, F), jnp.float32),
            pltpu.VMEM_SHARED((ZROWS, F), jnp.float32),
        ] + [pltpu.SemaphoreType.DMA] * (2 * NBUF),
    )
    def k(h_hbm, src_hbm, dst_hbm, z_hbm, out_hbm, sidx, didx, bufs, acc,
          *sems):
        sg = sems[:NBUF]
        ss = sems[NBUF:]
        c = lax.axis_index("c")
        s = lax.axis_index("s")
        w = c * NS + s
        pltpu.sync_copy(z_hbm, acc.at[pl.ds(s * RPS, RPS)])
        plsc.subcore_barrier()

        @pl.loop(0, NHALF)
        def _(half):
            pltpu.sync_copy(src_hbm.at[w].at[half], sidx)
            pltpu.sync_copy(dst_hbm.at[w].at[half], didx)

            @pl.loop(0, HALF, step=NBUF)
            def _(g):
                gd = [pltpu.async_copy(h_hbm.at[sidx.at[g + b]],
                                       bufs.at[b], sg[b])
                      for b in range(NBUF)]
                sd = []
                for b in range(NBUF):
                    gd[b].wait()
                    sd.append(pltpu.async_copy(
                        bufs.at[b], acc.at[didx.at[g + b]], ss[b], add=True))
                for b in range(NBUF):
                    sd[b].wait()

        plsc.subcore_barrier()
        pltpu.sync_copy(acc.at[pl.ds(s * RPS, RPS)],
                        out_hbm.at[c].at[pl.ds(s * RPS, RPS)])

    return k(h, src_w, dst_w, zeros_rows)


def _tc_mlp(x, parts, w1, b1, w2, b2):
    def body(x_ref, p_ref, w1_ref, b1_ref, w2_ref, b2_ref, o_ref):
        h = x_ref[...] + p_ref[0] + p_ref[1]
        a = jnp.dot(h, w1_ref[...], precision=_P,
                    preferred_element_type=jnp.float32) + b1_ref[...]
        a = jnp.maximum(a, 0.0)
        o = jnp.dot(a, w2_ref[...], precision=_P,
                    preferred_element_type=jnp.float32) + b2_ref[...]
        o_ref[...] = jnp.maximum(o, 0.0)

    return pl.pallas_call(
        body,
        grid=(N // BLK,),
        in_specs=[
            pl.BlockSpec((BLK, F), lambda i: (i, 0)),
            pl.BlockSpec((NC, BLK, F), lambda i: (0, i, 0)),
            pl.BlockSpec((F, F), lambda i: (0, 0)),
            pl.BlockSpec((1, F), lambda i: (0, 0)),
            pl.BlockSpec((F, F), lambda i: (0, 0)),
            pl.BlockSpec((1, F), lambda i: (0, 0)),
        ],
        out_specs=pl.BlockSpec((BLK, F), lambda i: (i, 0)),
        out_shape=jax.ShapeDtypeStruct((N, F), jnp.float32),
    )(x, parts, w1, b1.reshape(1, F), w2, b2.reshape(1, F))


def _tc_head(h1, h2, h3, lw1, lb1, lw2p, lb2p):
    def body(h1_ref, h2_ref, h3_ref, w1_ref, b1_ref, w2_ref, b2_ref, o_ref):
        t = (jnp.dot(h1_ref[...], w1_ref[0], precision=_P,
                     preferred_element_type=jnp.float32)
             + jnp.dot(h2_ref[...], w1_ref[1], precision=_P,
                       preferred_element_type=jnp.float32)
             + jnp.dot(h3_ref[...], w1_ref[2], precision=_P,
                       preferred_element_type=jnp.float32)) + b1_ref[...]
        t = jnp.maximum(t, 0.0)
        o = jnp.dot(t, w2_ref[...], precision=_P,
                    preferred_element_type=jnp.float32) + b2_ref[...]
        m = jnp.max(o, axis=1, keepdims=True)
        lse = jnp.log(jnp.sum(jnp.exp(o - m), axis=1, keepdims=True)) + m
        o_ref[...] = o - lse

    return pl.pallas_call(
        body,
        grid=(N // BLK,),
        in_specs=[
            pl.BlockSpec((BLK, F), lambda i: (i, 0)),
            pl.BlockSpec((BLK, F), lambda i: (i, 0)),
            pl.BlockSpec((BLK, F), lambda i: (i, 0)),
            pl.BlockSpec((3, F, 3 * F), lambda i: (0, 0, 0)),
            pl.BlockSpec((1, 3 * F), lambda i: (0, 0)),
            pl.BlockSpec((3 * F, F), lambda i: (0, 0)),
            pl.BlockSpec((1, F), lambda i: (0, 0)),
        ],
        out_specs=pl.BlockSpec((BLK, F), lambda i: (i, 0)),
        out_shape=jax.ShapeDtypeStruct((N, F), jnp.float32),
    )(h1, h2, h3, lw1, lb1.reshape(1, 3 * F), lw2p, lb2p.reshape(1, F))


def kernel(x, edge_index, gc1_w1, gc1_b1, gc1_w2, gc1_b2, gc2_w1, gc2_b1,
           gc2_w2, gc2_b2, gc3_w1, gc3_b1, gc3_w2, gc3_b2, lin1_w, lin1_b,
           lin2_w, lin2_b):
    src = edge_index[0]
    dst = edge_index[1]
    pad = EPAD - E
    src_w = jnp.concatenate(
        [src, jnp.zeros((pad,), src.dtype)]).reshape(NW, NHALF, HALF, CHUNK)
    dst_w = jnp.concatenate(
        [dst, jnp.full((pad,), N, dst.dtype)]).reshape(NW, NHALF, HALF, CHUNK)
    zeros_rows = jnp.zeros((RPS, F), jnp.float32)

    p1 = _sc_agg(x, src_w, dst_w, zeros_rows)
    h1 = _tc_mlp(x, p1, gc1_w1, gc1_b1, gc1_w2, gc1_b2)
    p2 = _sc_agg(h1, src_w, dst_w, zeros_rows)
    h2 = _tc_mlp(h1, p2, gc2_w1, gc2_b1, gc2_w2, gc2_b2)
    p3 = _sc_agg(h2, src_w, dst_w, zeros_rows)
    h3 = _tc_mlp(h2, p3, gc3_w1, gc3_b1, gc3_w2, gc3_b2)

    lw1 = lin1_w.reshape(3, F, 3 * F)
    lw2p = jnp.pad(lin2_w, ((0, 0), (0, F - NCLASS)))
    lb2p = jnp.concatenate(
        [lin2_b, jnp.full((F - NCLASS,), -1e30, jnp.float32)])
    out = _tc_head(h1, h2, h3, lw1, lin1_b, lw2p, lb2p)
    return out[:, :NCLASS]


# speedup vs baseline: 1.1007x; 1.1007x over previous
import functools

import jax
import jax.numpy as jnp
from jax import lax
from jax.experimental import pallas as pl
from jax.experimental.pallas import tpu as pltpu
from jax.experimental.pallas import tpu_sc as plsc

N = 10000
F = 128
E = 320000
NCLASS = 40

NC = 2
NS = 16
NW = NC * NS

CHUNK = 128
CPW = 80
NHALF = 2
HALF = CPW // NHALF
NBUF = 2
EPW = CPW * CHUNK
EPAD = NW * EPW
ZROWS = 10112
RPS = ZROWS // NS

BLK = 1000

_P = lax.Precision.HIGHEST


def _sc_agg(h, src_w, dst_w, zeros_rows):
    mesh = plsc.VectorSubcoreMesh(core_axis_name="c", subcore_axis_name="s")

    @functools.partial(
        pl.kernel,
        out_type=jax.ShapeDtypeStruct((NC, ZROWS, F), jnp.float32),
        mesh=mesh,
        scratch_types=[
            pltpu.VMEM((HALF, CHUNK), jnp.int32),
            pltpu.VMEM((HALF, CHUNK), jnp.int32),
            pltpu.VMEM((NBUF, CHUNK, F), jnp.float32),
            pltpu.VMEM_SHARED((ZROWS, F), jnp.float32),
        ] + [pltpu.SemaphoreType.DMA] * (2 * NBUF),
    )
    def k(h_hbm, src_hbm, dst_hbm, z_hbm, out_hbm, sidx, didx, bufs, acc,
          *sems):
        sg = sems[:NBUF]
        ss = sems[NBUF:]
        c = lax.axis_index("c")
        s = lax.axis_index("s")
        w = c * NS + s
        pltpu.sync_copy(z_hbm, acc.at[pl.ds(s * RPS, RPS)])
        plsc.subcore_barrier()

        @pl.loop(0, NHALF)
        def _(half):
            pltpu.sync_copy(src_hbm.at[w].at[half], sidx)
            pltpu.sync_copy(dst_hbm.at[w].at[half], didx)

            @pl.loop(0, HALF, step=NBUF)
            def _(g):
                gd = [pltpu.async_copy(h_hbm.at[sidx.at[g + b]],
                                       bufs.at[b], sg[b])
                      for b in range(NBUF)]
                sd = []
                for b in range(NBUF):
                    gd[b].wait()
                    sd.append(pltpu.async_copy(
                        bufs.at[b], acc.at[didx.at[g + b]], ss[b], add=True))
                for b in range(NBUF):
                    sd[b].wait()

        plsc.subcore_barrier()
        pltpu.sync_copy(acc.at[pl.ds(s * RPS, RPS)],
                        out_hbm.at[c].at[pl.ds(s * RPS, RPS)])

    return k(h, src_w, dst_w, zeros_rows)


def _tc_mlp(x, parts, w1, b1, w2, b2):
    def body(x_ref, p_ref, w1_ref, b1_ref, w2_ref, b2_ref, o_ref):
        h = x_ref[...] + p_ref[0] + p_ref[1]
        a = jnp.dot(h, w1_ref[...], precision=_P,
                    preferred_element_type=jnp.float32) + b1_ref[...]
        a = jnp.maximum(a, 0.0)
        o = jnp.dot(a, w2_ref[...], precision=_P,
                    preferred_element_type=jnp.float32) + b2_ref[...]
        o_ref[...] = jnp.maximum(o, 0.0)

    return pl.pallas_call(
        body,
        grid=(N // BLK,),
        in_specs=[
            pl.BlockSpec((BLK, F), lambda i: (i, 0)),
            pl.BlockSpec((NC, BLK, F), lambda i: (0, i, 0)),
            pl.BlockSpec((F, F), lambda i: (0, 0)),
            pl.BlockSpec((1, F), lambda i: (0, 0)),
            pl.BlockSpec((F, F), lambda i: (0, 0)),
            pl.BlockSpec((1, F), lambda i: (0, 0)),
        ],
        out_specs=pl.BlockSpec((BLK, F), lambda i: (i, 0)),
        out_shape=jax.ShapeDtypeStruct((N, F), jnp.float32),
    )(x, parts, w1, b1.reshape(1, F), w2, b2.reshape(1, F))


def _tc_head(h1, h2, h3, lw1, lb1, lw2p, lb2p):
    def body(h1_ref, h2_ref, h3_ref, w1_ref, b1_ref, w2_ref, b2_ref, o_ref):
        t = (jnp.dot(h1_ref[...], w1_ref[0], precision=_P,
                     preferred_element_type=jnp.float32)
             + jnp.dot(h2_ref[...], w1_ref[1], precision=_P,
                       preferred_element_type=jnp.float32)
             + jnp.dot(h3_ref[...], w1_ref[2], precision=_P,
                       preferred_element_type=jnp.float32)) + b1_ref[...]
        t = jnp.maximum(t, 0.0)
        o = jnp.dot(t, w2_ref[...], precision=_P,
                    preferred_element_type=jnp.float32) + b2_ref[...]
        m = jnp.max(o, axis=1, keepdims=True)
        lse = jnp.log(jnp.sum(jnp.exp(o - m), axis=1, keepdims=True)) + m
        o_ref[...] = o - lse

    return pl.pallas_call(
        body,
        grid=(N // BLK,),
        in_specs=[
            pl.BlockSpec((BLK, F), lambda i: (i, 0)),
            pl.BlockSpec((BLK, F), lambda i: (i, 0)),
            pl.BlockSpec((BLK, F), lambda i: (i, 0)),
            pl.BlockSpec((3, F, 3 * F), lambda i: (0, 0, 0)),
            pl.BlockSpec((1, 3 * F), lambda i: (0, 0)),
            pl.BlockSpec((3 * F, F), lambda i: (0, 0)),
            pl.BlockSpec((1, F), lambda i: (0, 0)),
        ],
        out_specs=pl.BlockSpec((BLK, F), lambda i: (i, 0)),
        out_shape=jax.ShapeDtypeStruct((N, F), jnp.float32),
    )(h1, h2, h3, lw1, lb1.reshape(1, 3 * F), lw2p, lb2p.reshape(1, F))


def kernel(x, edge_index, gc1_w1, gc1_b1, gc1_w2, gc1_b2, gc2_w1, gc2_b1,
           gc2_w2, gc2_b2, gc3_w1, gc3_b1, gc3_w2, gc3_b2, lin1_w, lin1_b,
           lin2_w, lin2_b):
    src = edge_index[0]
    dst = edge_index[1]
    pad = EPAD - E
    src_w = jnp.concatenate(
        [src, jnp.zeros((pad,), src.dtype)]).reshape(NW, NHALF, HALF, CHUNK)
    dst_w = jnp.concatenate(
        [dst, jnp.full((pad,), N, dst.dtype)]).reshape(NW, NHALF, HALF, CHUNK)
    zeros_rows = jnp.zeros((RPS, F), jnp.float32)

    p1 = _sc_agg(x, src_w, dst_w, zeros_rows)
    h1 = _tc_mlp(x, p1, gc1_w1, gc1_b1, gc1_w2, gc1_b2)
    p2 = _sc_agg(h1, src_w, dst_w, zeros_rows)
    h2 = _tc_mlp(h1, p2, gc2_w1, gc2_b1, gc2_w2, gc2_b2)
    p3 = _sc_agg(h2, src_w, dst_w, zeros_rows)
    h3 = _tc_mlp(h2, p3, gc3_w1, gc3_b1, gc3_w2, gc3_b2)

    lw1 = lin1_w.reshape(3, F, 3 * F)
    lw2p = jnp.pad(lin2_w, ((0, 0), (0, F - NCLASS)))
    lb2p = jnp.concatenate(
        [lin2_b, jnp.full((F - NCLASS,), -1e30, jnp.float32)])
    out = _tc_head(h1, h2, h3, lw1, lin1_b, lw2p, lb2p)
    return out[:, :NCLASS]


# speedup vs baseline: 1.1071x; 1.0058x over previous
import functools

import jax
import jax.numpy as jnp
from jax import lax
from jax.experimental import pallas as pl
from jax.experimental.pallas import tpu as pltpu
from jax.experimental.pallas import tpu_sc as plsc

N = 10000
F = 128
E = 320000
NCLASS = 40

NC = 2
NS = 16
NW = NC * NS

CHUNK = 128
CPW = 80
NHALF = 2
HALF = CPW // NHALF
NBUF = 2
EPW = CPW * CHUNK
EPAD = NW * EPW
ZROWS = 10112
RPS = ZROWS // NS

BLK = 1000

_P = lax.Precision.HIGHEST


def _sc_agg(h, src_w, dst_w, zeros_rows):
    mesh = plsc.VectorSubcoreMesh(core_axis_name="c", subcore_axis_name="s")

    @functools.partial(
        pl.kernel,
        out_type=jax.ShapeDtypeStruct((NC, ZROWS, F), jnp.float32),
        mesh=mesh,
        scratch_types=[
            pltpu.VMEM((HALF, CHUNK), jnp.int32),
            pltpu.VMEM((HALF, CHUNK), jnp.int32),
            pltpu.VMEM((NBUF, CHUNK, F), jnp.float32),
            pltpu.VMEM_SHARED((ZROWS, F), jnp.float32),
        ] + [pltpu.SemaphoreType.DMA] * (2 * NBUF),
    )
    def k(h_hbm, src_hbm, dst_hbm, z_hbm, out_hbm, sidx, didx, bufs, acc,
          *sems):
        sg = sems[:NBUF]
        ss = sems[NBUF:]
        c = lax.axis_index("c")
        s = lax.axis_index("s")
        w = c * NS + s
        pltpu.sync_copy(z_hbm, acc.at[pl.ds(s * RPS, RPS)])
        plsc.subcore_barrier()

        @pl.loop(0, NHALF)
        def _(half):
            pltpu.sync_copy(src_hbm.at[w].at[half], sidx)
            pltpu.sync_copy(dst_hbm.at[w].at[half], didx)

            @pl.loop(0, HALF, step=2 * NBUF)
            def _(g):
                def gath(i, b):
                    return pltpu.async_copy(h_hbm.at[sidx.at[g + i]],
                                            bufs.at[b], sg[b])

                def scat(i, b):
                    return pltpu.async_copy(
                        bufs.at[b], acc.at[didx.at[g + i]], ss[b], add=True)

                d0, d1 = gath(0, 0), gath(1, 1)
                d0.wait()
                s0 = scat(0, 0)
                d1.wait()
                s1 = scat(1, 1)
                s0.wait()
                d2 = gath(2, 0)
                s1.wait()
                d3 = gath(3, 1)
                d2.wait()
                s2 = scat(2, 0)
                d3.wait()
                s3 = scat(3, 1)
                s2.wait()
                s3.wait()

        plsc.subcore_barrier()
        pltpu.sync_copy(acc.at[pl.ds(s * RPS, RPS)],
                        out_hbm.at[c].at[pl.ds(s * RPS, RPS)])

    return k(h, src_w, dst_w, zeros_rows)


def _tc_mlp(x, parts, w1, b1, w2, b2):
    def body(x_ref, p_ref, w1_ref, b1_ref, w2_ref, b2_ref, o_ref):
        h = x_ref[...] + p_ref[0] + p_ref[1]
        a = jnp.dot(h, w1_ref[...], precision=_P,
                    preferred_element_type=jnp.float32) + b1_ref[...]
        a = jnp.maximum(a, 0.0)
        o = jnp.dot(a, w2_ref[...], precision=_P,
                    preferred_element_type=jnp.float32) + b2_ref[...]
        o_ref[...] = jnp.maximum(o, 0.0)

    return pl.pallas_call(
        body,
        grid=(N // BLK,),
        in_specs=[
            pl.BlockSpec((BLK, F), lambda i: (i, 0)),
            pl.BlockSpec((NC, BLK, F), lambda i: (0, i, 0)),
            pl.BlockSpec((F, F), lambda i: (0, 0)),
            pl.BlockSpec((1, F), lambda i: (0, 0)),
            pl.BlockSpec((F, F), lambda i: (0, 0)),
            pl.BlockSpec((1, F), lambda i: (0, 0)),
        ],
        out_specs=pl.BlockSpec((BLK, F), lambda i: (i, 0)),
        out_shape=jax.ShapeDtypeStruct((N, F), jnp.float32),
    )(x, parts, w1, b1.reshape(1, F), w2, b2.reshape(1, F))


def _tc_head(h1, h2, h3, lw1, lb1, lw2p, lb2p):
    def body(h1_ref, h2_ref, h3_ref, w1_ref, b1_ref, w2_ref, b2_ref, o_ref):
        t = (jnp.dot(h1_ref[...], w1_ref[0], precision=_P,
                     preferred_element_type=jnp.float32)
             + jnp.dot(h2_ref[...], w1_ref[1], precision=_P,
                       preferred_element_type=jnp.float32)
             + jnp.dot(h3_ref[...], w1_ref[2], precision=_P,
                       preferred_element_type=jnp.float32)) + b1_ref[...]
        t = jnp.maximum(t, 0.0)
        o = jnp.dot(t, w2_ref[...], precision=_P,
                    preferred_element_type=jnp.float32) + b2_ref[...]
        m = jnp.max(o, axis=1, keepdims=True)
        lse = jnp.log(jnp.sum(jnp.exp(o - m), axis=1, keepdims=True)) + m
        o_ref[...] = o - lse

    return pl.pallas_call(
        body,
        grid=(N // BLK,),
        in_specs=[
            pl.BlockSpec((BLK, F), lambda i: (i, 0)),
            pl.BlockSpec((BLK, F), lambda i: (i, 0)),
            pl.BlockSpec((BLK, F), lambda i: (i, 0)),
            pl.BlockSpec((3, F, 3 * F), lambda i: (0, 0, 0)),
            pl.BlockSpec((1, 3 * F), lambda i: (0, 0)),
            pl.BlockSpec((3 * F, F), lambda i: (0, 0)),
            pl.BlockSpec((1, F), lambda i: (0, 0)),
        ],
        out_specs=pl.BlockSpec((BLK, F), lambda i: (i, 0)),
        out_shape=jax.ShapeDtypeStruct((N, F), jnp.float32),
    )(h1, h2, h3, lw1, lb1.reshape(1, 3 * F), lw2p, lb2p.reshape(1, F))


def kernel(x, edge_index, gc1_w1, gc1_b1, gc1_w2, gc1_b2, gc2_w1, gc2_b1,
           gc2_w2, gc2_b2, gc3_w1, gc3_b1, gc3_w2, gc3_b2, lin1_w, lin1_b,
           lin2_w, lin2_b):
    src = edge_index[0]
    dst = edge_index[1]
    pad = EPAD - E
    src_w = jnp.concatenate(
        [src, jnp.zeros((pad,), src.dtype)]).reshape(NW, NHALF, HALF, CHUNK)
    dst_w = jnp.concatenate(
        [dst, jnp.full((pad,), N, dst.dtype)]).reshape(NW, NHALF, HALF, CHUNK)
    zeros_rows = jnp.zeros((RPS, F), jnp.float32)

    p1 = _sc_agg(x, src_w, dst_w, zeros_rows)
    h1 = _tc_mlp(x, p1, gc1_w1, gc1_b1, gc1_w2, gc1_b2)
    p2 = _sc_agg(h1, src_w, dst_w, zeros_rows)
    h2 = _tc_mlp(h1, p2, gc2_w1, gc2_b1, gc2_w2, gc2_b2)
    p3 = _sc_agg(h2, src_w, dst_w, zeros_rows)
    h3 = _tc_mlp(h2, p3, gc3_w1, gc3_b1, gc3_w2, gc3_b2)

    lw1 = lin1_w.reshape(3, F, 3 * F)
    lw2p = jnp.pad(lin2_w, ((0, 0), (0, F - NCLASS)))
    lb2p = jnp.concatenate(
        [lin2_b, jnp.full((F - NCLASS,), -1e30, jnp.float32)])
    out = _tc_head(h1, h2, h3, lw1, lin1_b, lw2p, lb2p)
    return out[:, :NCLASS]
